# baseline (speedup 1.0000x reference)
import jax
import jax.numpy as jnp
from jax.experimental import pallas as pl
from jax.experimental.pallas import tpu as pltpu


def _gcn_kernel(x_ref, w_ref, b_ref, adj0_ref, adj1_ref, o_ref, s_ref):
    @pl.when(pl.program_id(0) == 0)
    def _support():
        s_ref[...] = jnp.dot(x_ref[...], w_ref[...],
                             preferred_element_type=jnp.float32)

    bm = adj0_ref.shape[0]
    p0 = jnp.dot(adj0_ref[...], s_ref[...], preferred_element_type=jnp.float32)
    o_ref[:bm, :] = jnp.maximum(p0 + b_ref[...], 0.0)
    p1 = jnp.dot(adj1_ref[...], s_ref[...], preferred_element_type=jnp.float32)
    o_ref[bm:, :] = jnp.maximum(p1 + b_ref[...], 0.0)


def kernel(x, adj, W, b):
    n, nfeat = x.shape
    nout = W.shape[1]

    bm = 200
    m_blocks = n // (2 * bm)

    out = pl.pallas_call(
        _gcn_kernel,
        grid=(m_blocks,),
        in_specs=[
            pl.BlockSpec((n, nfeat), lambda i: (0, 0)),
            pl.BlockSpec((nfeat, nout), lambda i: (0, 0)),
            pl.BlockSpec((1, nout), lambda i: (0, 0)),
            pl.BlockSpec((bm, n), lambda i: (2 * i, 0)),
            pl.BlockSpec((bm, n), lambda i: (2 * i + 1, 0)),
        ],
        out_specs=pl.BlockSpec((2 * bm, nout), lambda i: (i, 0)),
        out_shape=jax.ShapeDtypeStruct((n, nout), jnp.float32),
        scratch_shapes=[pltpu.VMEM((n, nout), jnp.float32)],
    )(x, W, b.reshape(1, nout), adj, adj)
    return out
